# baseline (speedup 1.0000x reference)
import functools

import jax
import jax.numpy as jnp
from jax.experimental import pallas as pl

_TT = (((1,), (1,)), ((), ()))


def _mlp_body(x_ref, w1_ref, w2_ref, w3_ref, out_ref):
    h1 = jnp.maximum(
        jax.lax.dot_general(w1_ref[...], x_ref[...], _TT,
                            preferred_element_type=jnp.float32), 0.0
    )
    h2 = jnp.maximum(
        jnp.dot(w2_ref[...], h1, preferred_element_type=jnp.float32), 0.0
    )
    out_ref[...] = jnp.dot(w3_ref[...], h2,
                           preferred_element_type=jnp.float32)


@functools.partial(jax.jit, static_argnames=("tile_b",))
def _mlp(x, W1, W2, W3, tile_b=2048):
    b, d_in = x.shape
    h = W1.shape[0]
    n_out = W3.shape[0]
    d_pad = (d_in + 127) // 128 * 128
    xp = jnp.pad(x, ((0, 0), (0, d_pad - d_in)))
    w1p = jnp.pad(W1, ((0, 0), (0, d_pad - d_in)))
    grid = (b // tile_b,)
    out_t = pl.pallas_call(
        _mlp_body,
        grid=grid,
        in_specs=[
            pl.BlockSpec((tile_b, d_pad), lambda i: (i, 0)),
            pl.BlockSpec((h, d_pad), lambda i: (0, 0)),
            pl.BlockSpec((h, h), lambda i: (0, 0)),
            pl.BlockSpec((n_out, h), lambda i: (0, 0)),
        ],
        out_specs=pl.BlockSpec((n_out, tile_b), lambda i: (0, i)),
        out_shape=jax.ShapeDtypeStruct((n_out, b), jnp.float32),
    )(xp, w1p, W2, W3)
    return out_t.T


def kernel(x, W1, W2, W3):
    return _mlp(x, W1, W2, W3)


# speedup vs baseline: 3.5663x; 3.5663x over previous
import functools

import jax
import jax.numpy as jnp
from jax.experimental import pallas as pl
from jax.experimental.pallas import tpu as pltpu

_TLHS = (((0,), (0,)), ((), ()))


def _make_body(tile_b):
    def _mlp_body(xt_hbm, w1t_ref, w2_ref, w3_ref, out_ref, xbuf, sem):
        i = pl.program_id(0)
        n = pl.num_programs(0)
        slot = jax.lax.rem(i, 2)
        nxt = jax.lax.rem(i + 1, 2)

        @pl.when(i == 0)
        def _():
            pltpu.make_async_copy(
                xt_hbm.at[:, pl.ds(0, tile_b)], xbuf.at[0], sem.at[0]
            ).start()

        @pl.when(i + 1 < n)
        def _():
            pltpu.make_async_copy(
                xt_hbm.at[:, pl.ds((i + 1) * tile_b, tile_b)], xbuf.at[nxt],
                sem.at[nxt],
            ).start()

        pltpu.make_async_copy(
            xt_hbm.at[:, pl.ds(i * tile_b, tile_b)], xbuf.at[slot], sem.at[slot]
        ).wait()

        h1 = jnp.maximum(
            jax.lax.dot_general(w1t_ref[...], xbuf[slot], _TLHS,
                                preferred_element_type=jnp.float32), 0.0
        )
        h2 = jnp.maximum(
            jnp.dot(w2_ref[...], h1, preferred_element_type=jnp.float32), 0.0
        )
        out_ref[...] = jnp.dot(w3_ref[...], h2,
                               preferred_element_type=jnp.float32)

    return _mlp_body


@functools.partial(jax.jit, static_argnames=("tile_b",))
def _mlp(x, W1, W2, W3, tile_b=2048):
    b, d_in = x.shape
    h = W1.shape[0]
    n_out = W3.shape[0]
    xt = x.T
    w1t = W1.T
    grid = (b // tile_b,)
    out_t = pl.pallas_call(
        _make_body(tile_b),
        grid=grid,
        in_specs=[
            pl.BlockSpec(memory_space=pltpu.MemorySpace.HBM),
            pl.BlockSpec((d_in, h), lambda i: (0, 0)),
            pl.BlockSpec((h, h), lambda i: (0, 0)),
            pl.BlockSpec((n_out, h), lambda i: (0, 0)),
        ],
        out_specs=pl.BlockSpec((n_out, tile_b), lambda i: (0, i)),
        out_shape=jax.ShapeDtypeStruct((n_out, b), jnp.float32),
        scratch_shapes=[
            pltpu.MemorySpace.VMEM((2, d_in, tile_b), jnp.float32),
            pltpu.SemaphoreType.DMA((2,)),
        ],
    )(xt, w1t, W2, W3)
    return out_t.T


def kernel(x, W1, W2, W3):
    return _mlp(x, W1, W2, W3)
